# baseline (speedup 1.0000x reference)
import functools

import jax
import jax.numpy as jnp
from jax import lax
from jax.experimental import pallas as pl
from jax.experimental.pallas import tpu as pltpu
from jax.experimental.pallas import tpu_sc as plsc

_NC, _NS, _L = 2, 16, 16
_NW = _NC * _NS


def _make_sc_rev(R, F):
    rows_per_w = R // _NW
    CH = 8
    n_chunks = rows_per_w // CH
    n_vec = F // _L
    mesh = plsc.VectorSubcoreMesh(core_axis_name="c", subcore_axis_name="s")

    @functools.partial(
        pl.kernel,
        mesh=mesh,
        out_type=jax.ShapeDtypeStruct((R, F), jnp.float32),
        scratch_types=[
            pltpu.VMEM((CH, F), jnp.float32),
            pltpu.VMEM((CH, F), jnp.float32),
            pltpu.VMEM((CH, F), jnp.float32),
            pltpu.VMEM((CH, F), jnp.float32),
            pltpu.SemaphoreType.DMA,
            pltpu.SemaphoreType.DMA,
            pltpu.SemaphoreType.DMA,
            pltpu.SemaphoreType.DMA,
        ],
    )
    def _sc_rev(x_hbm, o_hbm, in0, in1, out0, out1, si0, si1, so0, so1):
        wid = lax.axis_index("s") * _NC + lax.axis_index("c")
        base = wid * rows_per_w
        ins = (in0, in1)
        outs = (out0, out1)
        sis = (si0, si1)
        sos = (so0, so1)

        def in_copy(ci, b):
            return pltpu.make_async_copy(
                x_hbm.at[pl.ds(base + ci * CH, CH)], ins[b], sis[b]
            )

        def out_copy(ci, b):
            return pltpu.make_async_copy(
                outs[b], o_hbm.at[pl.ds(base + ci * CH, CH)], sos[b]
            )

        def compute(b):
            bi = ins[b]
            bo = outs[b]

            @plsc.parallel_loop(0, CH * n_vec, 1, unroll=16)
            def _(i):
                r = lax.shift_right_logical(i, 7)
                c = lax.bitwise_and(i, n_vec - 1)
                v = bi[r, pl.ds((n_vec - 1 - c) * _L, _L)]
                bo[r, pl.ds(c * _L, _L)] = lax.rev(v, (0,))

        in_copy(0, 0).start()
        in_copy(1, 1).start()

        def chunk_body(ci, carry):
            def do(bb):
                in_copy(ci, bb).wait()

                @pl.when(ci >= 2)
                def _():
                    out_copy(ci - 2, bb).wait()

                compute(bb)
                out_copy(ci, bb).start()

                @pl.when(ci + 2 < n_chunks)
                def _():
                    in_copy(ci + 2, bb).start()

            lax.cond(lax.rem(ci, 2) == 0, lambda: do(0), lambda: do(1))
            return carry

        lax.fori_loop(0, n_chunks, chunk_body, 0)
        out_copy(n_chunks - 2, 0).wait()
        out_copy(n_chunks - 1, 1).wait()

    return _sc_rev


def kernel(inputs):
    B, S, F = inputs.shape
    R = B * S
    x = inputs.reshape(R, F)
    out = _make_sc_rev(R, F)(x)
    return out.reshape(B, S, F)


# speedup vs baseline: 1.0242x; 1.0242x over previous
import functools

import jax
import jax.numpy as jnp
from jax import lax
from jax.experimental import pallas as pl
from jax.experimental.pallas import tpu as pltpu
from jax.experimental.pallas import tpu_sc as plsc

_NC, _NS, _L = 2, 16, 16
_NW = _NC * _NS


def _make_sc_rev(R, F):
    rows_per_w = R // _NW
    CH = 8
    n_chunks = rows_per_w // CH
    n_vec = F // _L
    mesh = plsc.VectorSubcoreMesh(core_axis_name="c", subcore_axis_name="s")

    @functools.partial(
        pl.kernel,
        mesh=mesh,
        out_type=jax.ShapeDtypeStruct((R, F), jnp.float32),
        scratch_types=[
            pltpu.VMEM((CH, F), jnp.float32),
            pltpu.VMEM((CH, F), jnp.float32),
            pltpu.VMEM((CH, F), jnp.float32),
            pltpu.VMEM((CH, F), jnp.float32),
            pltpu.VMEM((CH, F), jnp.float32),
            pltpu.VMEM((CH, F), jnp.float32),
            pltpu.SemaphoreType.DMA,
            pltpu.SemaphoreType.DMA,
            pltpu.SemaphoreType.DMA,
            pltpu.SemaphoreType.DMA,
            pltpu.SemaphoreType.DMA,
            pltpu.SemaphoreType.DMA,
        ],
    )
    def _sc_rev(
        x_hbm, o_hbm, in0, in1, in2, out0, out1, out2, si0, si1, si2, so0, so1, so2
    ):
        wid = lax.axis_index("s") * _NC + lax.axis_index("c")
        base = wid * rows_per_w
        ins = (in0, in1, in2)
        outs = (out0, out1, out2)
        sis = (si0, si1, si2)
        sos = (so0, so1, so2)

        def in_copy(ci, b):
            return pltpu.make_async_copy(
                x_hbm.at[pl.ds(base + ci * CH, CH)], ins[b], sis[b]
            )

        def out_copy(ci, b):
            return pltpu.make_async_copy(
                outs[b], o_hbm.at[pl.ds(base + ci * CH, CH)], sos[b]
            )

        def compute(b):
            bi = ins[b]
            bo = outs[b]

            @plsc.parallel_loop(0, CH * n_vec, 1, unroll=16)
            def _(i):
                r = lax.shift_right_logical(i, 7)
                c = lax.bitwise_and(i, n_vec - 1)
                v = bi[r, pl.ds((n_vec - 1 - c) * _L, _L)]
                bo[r, pl.ds(c * _L, _L)] = lax.rev(v, (0,))

        NB = 3
        in_copy(0, 0).start()
        in_copy(1, 1).start()
        in_copy(2, 2).start()

        def chunk_body(ci, carry):
            def do(bb):
                in_copy(ci, bb).wait()

                @pl.when(ci >= NB)
                def _():
                    out_copy(ci - NB, bb).wait()

                compute(bb)
                out_copy(ci, bb).start()

                @pl.when(ci + NB < n_chunks)
                def _():
                    in_copy(ci + NB, bb).start()

            lax.switch(lax.rem(ci, NB), [lambda: do(0), lambda: do(1), lambda: do(2)])
            return carry

        lax.fori_loop(0, n_chunks, chunk_body, 0)
        out_copy(n_chunks - 3, (n_chunks - 3) % 3).wait()
        out_copy(n_chunks - 2, (n_chunks - 2) % 3).wait()
        out_copy(n_chunks - 1, (n_chunks - 1) % 3).wait()

    return _sc_rev


def kernel(inputs):
    B, S, F = inputs.shape
    R = B * S
    x = inputs.reshape(R, F)
    out = _make_sc_rev(R, F)(x)
    return out.reshape(B, S, F)
